# baseline (speedup 1.0000x reference)
import jax
import jax.numpy as jnp
from jax.experimental import pallas as pl
from jax.experimental.pallas import tpu as pltpu

_TEMP = 0.05
_INV_TEMP = 1.0 / _TEMP
_C = _INV_TEMP * 1.4426950408889634
_BIG = 1e30

_BM = 512
_BR = 2048
_BC = 2048


def _selu(x):
    scale = 1.0507009873554805
    alpha = 1.6732632423543772
    return scale * jnp.where(x > 0, x, alpha * (jnp.exp(x) - 1.0))


def _mlp_kernel(z_ref, w1, b1, w2, b2, w3, b3, w4, b4, w5, b5, out_ref):
    h = _selu(jnp.dot(z_ref[...], w1[...], preferred_element_type=jnp.float32) + b1[...])
    h = _selu(jnp.dot(h, w2[...], preferred_element_type=jnp.float32) + b2[...])
    h = _selu(jnp.dot(h, w3[...], preferred_element_type=jnp.float32) + b3[...])
    h = _selu(jnp.dot(h, w4[...], preferred_element_type=jnp.float32) + b4[...])
    out_ref[...] = jnp.dot(h, w5[...], preferred_element_type=jnp.float32) + b5[...]


def _mlp(z, W1, b1, W2, b2, W3, b3, W4, b4, W5, b5):
    n = z.shape[0]
    full = lambda i: (0, 0)
    return pl.pallas_call(
        _mlp_kernel,
        out_shape=jax.ShapeDtypeStruct((n, 2), jnp.float32),
        grid=(n // _BM,),
        in_specs=[
            pl.BlockSpec((_BM, z.shape[1]), lambda i: (i, 0)),
            pl.BlockSpec(W1.shape, full), pl.BlockSpec((1, b1.shape[1]), full),
            pl.BlockSpec(W2.shape, full), pl.BlockSpec((1, b2.shape[1]), full),
            pl.BlockSpec(W3.shape, full), pl.BlockSpec((1, b3.shape[1]), full),
            pl.BlockSpec(W4.shape, full), pl.BlockSpec((1, b4.shape[1]), full),
            pl.BlockSpec(W5.shape, full), pl.BlockSpec((1, b5.shape[1]), full),
        ],
        out_specs=pl.BlockSpec((_BM, 2), lambda i: (i, 0)),
        compiler_params=pltpu.CompilerParams(
            dimension_semantics=("parallel",),
        ),
        name="drift_mlp",
    )(z, W1, b1, W2, b2, W3, b3, W4, b4, W5, b5)


def _energy_kernel(genL_ref, genR_ref, posR_ref, out_ref):
    i0 = pl.program_id(0) * _BR
    n = genR_ref.shape[1]
    n_tiles = n // _BC
    reps = _BC // 128

    def wide(v):
        return pltpu.repeat(jnp.broadcast_to(v, (_BR, 128)), reps, 1)

    gl = genL_ref[...]

    def body(j, carry):
        mp, sp, mn, sn = carry
        joff = pl.multiple_of(j * _BC, _BC)
        ptile = posR_ref[:, pl.ds(joff, _BC)]
        qtile = genR_ref[:, pl.ds(joff, _BC)]

        d2p = jnp.maximum(
            jnp.dot(gl, ptile, preferred_element_type=jnp.float32), 1e-12)
        dp = d2p * jax.lax.rsqrt(d2p)
        mp_new = jnp.minimum(mp, jnp.min(dp, axis=1, keepdims=True))
        ep = jnp.exp2((wide(mp_new) - dp) * _C)
        sp = sp * jnp.exp2((mp_new - mp) * _C) + jnp.sum(ep, axis=1, keepdims=True)

        d2n = jnp.maximum(
            jnp.dot(gl, qtile, preferred_element_type=jnp.float32), 1e-12)
        dn = d2n * jax.lax.rsqrt(d2n)

        rel = (i0 - joff) + jax.lax.broadcasted_iota(jnp.int32, (_BR, 1), 0)
        lanes = jax.lax.broadcasted_iota(jnp.int32, (_BR, _BC), 1)
        dn = jnp.where(rel == lanes, _BIG, dn)
        mn_new = jnp.minimum(mn, jnp.min(dn, axis=1, keepdims=True))
        en = jnp.exp2((wide(mn_new) - dn) * _C)
        sn = sn * jnp.exp2((mn_new - mn) * _C) + jnp.sum(en, axis=1, keepdims=True)

        return mp_new, sp, mn_new, sn

    init = (jnp.full((_BR, 1), _BIG, jnp.float32), jnp.zeros((_BR, 1), jnp.float32),
            jnp.full((_BR, 1), _BIG, jnp.float32), jnp.zeros((_BR, 1), jnp.float32))
    mp, sp, mn, sn = jax.lax.fori_loop(0, n_tiles, body, init)

    out_ref[...] = (mp - mn) + _TEMP * (jnp.log(sn) - jnp.log(sp))


def _energy(genL, genR, posR):
    n = genL.shape[0]
    return pl.pallas_call(
        _energy_kernel,
        out_shape=jax.ShapeDtypeStruct((n, 1), jnp.float32),
        grid=(n // _BR,),
        in_specs=[
            pl.BlockSpec((_BR, 4), lambda i: (i, 0)),
            pl.BlockSpec((4, n), lambda i: (0, 0)),
            pl.BlockSpec((4, n), lambda i: (0, 0)),
        ],
        out_specs=pl.BlockSpec((_BR, 1), lambda i: (i, 0)),
        compiler_params=pltpu.CompilerParams(
            dimension_semantics=("parallel",),
        ),
        name="drift_energy",
    )(genL, genR, posR)


def kernel(pos, z, W1, b1, W2, b2, W3, b3, W4, b4, W5, b5):
    n = pos.shape[0]
    gen = _mlp(z, W1, b1.reshape(1, -1), W2, b2.reshape(1, -1),
               W3, b3.reshape(1, -1), W4, b4.reshape(1, -1),
               W5, b5.reshape(1, -1))
    ones = jnp.ones((n, 1), jnp.float32)
    g2 = jnp.sum(gen * gen, axis=1, keepdims=True)
    p2 = jnp.sum(pos * pos, axis=1, keepdims=True)
    genL = jnp.concatenate([gen, g2, ones], axis=1)
    genR = jnp.concatenate([-2.0 * gen, ones, g2], axis=1).T
    posR = jnp.concatenate([-2.0 * pos, ones, p2], axis=1).T
    energy = _energy(genL, genR, posR)
    return energy[:, 0]


# speedup vs baseline: 1.0251x; 1.0251x over previous
import jax
import jax.numpy as jnp
from jax.experimental import pallas as pl
from jax.experimental.pallas import tpu as pltpu

_TEMP = 0.05
_INV_TEMP = 1.0 / _TEMP
_C = _INV_TEMP * 1.4426950408889634
_BIG = 1e30

_BM = 512
_BR = 2048
_BC = 4096


def _selu(x):
    scale = 1.0507009873554805
    alpha = 1.6732632423543772
    return scale * jnp.where(x > 0, x, alpha * (jnp.exp(x) - 1.0))


def _mlp_kernel(z_ref, w1, b1, w2, b2, w3, b3, w4, b4, w5, b5, out_ref):
    h = _selu(jnp.dot(z_ref[...], w1[...], preferred_element_type=jnp.float32) + b1[...])
    h = _selu(jnp.dot(h, w2[...], preferred_element_type=jnp.float32) + b2[...])
    h = _selu(jnp.dot(h, w3[...], preferred_element_type=jnp.float32) + b3[...])
    h = _selu(jnp.dot(h, w4[...], preferred_element_type=jnp.float32) + b4[...])
    out_ref[...] = jnp.dot(h, w5[...], preferred_element_type=jnp.float32) + b5[...]


def _mlp(z, W1, b1, W2, b2, W3, b3, W4, b4, W5, b5):
    n = z.shape[0]
    full = lambda i: (0, 0)
    return pl.pallas_call(
        _mlp_kernel,
        out_shape=jax.ShapeDtypeStruct((n, 2), jnp.float32),
        grid=(n // _BM,),
        in_specs=[
            pl.BlockSpec((_BM, z.shape[1]), lambda i: (i, 0)),
            pl.BlockSpec(W1.shape, full), pl.BlockSpec((1, b1.shape[1]), full),
            pl.BlockSpec(W2.shape, full), pl.BlockSpec((1, b2.shape[1]), full),
            pl.BlockSpec(W3.shape, full), pl.BlockSpec((1, b3.shape[1]), full),
            pl.BlockSpec(W4.shape, full), pl.BlockSpec((1, b4.shape[1]), full),
            pl.BlockSpec(W5.shape, full), pl.BlockSpec((1, b5.shape[1]), full),
        ],
        out_specs=pl.BlockSpec((_BM, 2), lambda i: (i, 0)),
        compiler_params=pltpu.CompilerParams(
            dimension_semantics=("parallel",),
        ),
        name="drift_mlp",
    )(z, W1, b1, W2, b2, W3, b3, W4, b4, W5, b5)


def _energy_kernel(genL_ref, genR_ref, posR_ref, out_ref):
    i0 = pl.program_id(0) * _BR
    n = genR_ref.shape[1]
    n_tiles = n // _BC
    reps = _BC // 128

    def wide(v):
        return pltpu.repeat(jnp.broadcast_to(v, (_BR, 128)), reps, 1)

    gl = genL_ref[...]

    def body(j, carry):
        mp, sp, mn, sn = carry
        joff = pl.multiple_of(j * _BC, _BC)
        ptile = posR_ref[:, pl.ds(joff, _BC)]
        qtile = genR_ref[:, pl.ds(joff, _BC)]

        d2p = jnp.maximum(
            jnp.dot(gl, ptile, preferred_element_type=jnp.float32), 1e-12)
        dp = d2p * jax.lax.rsqrt(d2p)
        mp_new = jnp.minimum(mp, jnp.min(dp, axis=1, keepdims=True))
        ep = jnp.exp2((wide(mp_new) - dp) * _C)
        sp = sp * jnp.exp2((mp_new - mp) * _C) + jnp.sum(ep, axis=1, keepdims=True)

        d2n = jnp.maximum(
            jnp.dot(gl, qtile, preferred_element_type=jnp.float32), 1e-12)
        dn = d2n * jax.lax.rsqrt(d2n)

        rel = (i0 - joff) + jax.lax.broadcasted_iota(jnp.int32, (_BR, 1), 0)
        lanes = jax.lax.broadcasted_iota(jnp.int32, (_BR, _BC), 1)
        dn = jnp.where(rel == lanes, _BIG, dn)
        mn_new = jnp.minimum(mn, jnp.min(dn, axis=1, keepdims=True))
        en = jnp.exp2((wide(mn_new) - dn) * _C)
        sn = sn * jnp.exp2((mn_new - mn) * _C) + jnp.sum(en, axis=1, keepdims=True)

        return mp_new, sp, mn_new, sn

    init = (jnp.full((_BR, 1), _BIG, jnp.float32), jnp.zeros((_BR, 1), jnp.float32),
            jnp.full((_BR, 1), _BIG, jnp.float32), jnp.zeros((_BR, 1), jnp.float32))
    mp, sp, mn, sn = jax.lax.fori_loop(0, n_tiles, body, init)

    out_ref[...] = (mp - mn) + _TEMP * (jnp.log(sn) - jnp.log(sp))


def _energy(genL, genR, posR):
    n = genL.shape[0]
    return pl.pallas_call(
        _energy_kernel,
        out_shape=jax.ShapeDtypeStruct((n, 1), jnp.float32),
        grid=(n // _BR,),
        in_specs=[
            pl.BlockSpec((_BR, 4), lambda i: (i, 0)),
            pl.BlockSpec((4, n), lambda i: (0, 0)),
            pl.BlockSpec((4, n), lambda i: (0, 0)),
        ],
        out_specs=pl.BlockSpec((_BR, 1), lambda i: (i, 0)),
        compiler_params=pltpu.CompilerParams(
            dimension_semantics=("parallel",),
        ),
        name="drift_energy",
    )(genL, genR, posR)


def kernel(pos, z, W1, b1, W2, b2, W3, b3, W4, b4, W5, b5):
    n = pos.shape[0]
    gen = _mlp(z, W1, b1.reshape(1, -1), W2, b2.reshape(1, -1),
               W3, b3.reshape(1, -1), W4, b4.reshape(1, -1),
               W5, b5.reshape(1, -1))
    ones = jnp.ones((n, 1), jnp.float32)
    g2 = jnp.sum(gen * gen, axis=1, keepdims=True)
    p2 = jnp.sum(pos * pos, axis=1, keepdims=True)
    genL = jnp.concatenate([gen, g2, ones], axis=1)
    genR = jnp.concatenate([-2.0 * gen, ones, g2], axis=1).T
    posR = jnp.concatenate([-2.0 * pos, ones, p2], axis=1).T
    energy = _energy(genL, genR, posR)
    return energy[:, 0]
